# baseline (speedup 1.0000x reference)
import jax
import jax.numpy as jnp
from jax.experimental import pallas as pl
from jax.experimental.pallas import tpu as pltpu

_T = 16


def _body(m_ref, head_ref, o_ref):
    C = head_ref.shape[-1]
    v = head_ref[...]
    cls = v[:, 0, :]
    shifted_f = jnp.dot(m_ref[0], cls, preferred_element_type=jnp.float32)
    shifted_p = jnp.dot(m_ref[1], cls, preferred_element_type=jnp.float32)
    c_idx = jax.lax.broadcasted_iota(jnp.int32, cls.shape, 1)
    new_cls = cls + jnp.where(c_idx < C // 2, shifted_f, shifted_p)
    n_idx = jax.lax.broadcasted_iota(jnp.int32, v.shape, 1)
    o_ref[...] = jnp.where(n_idx == 0, new_cls[:, None, :], v)


def kernel(x, past_shift_raw, future_shift_raw):
    B_T, N, C = x.shape

    def _shift(raw):
        return jnp.round(jax.nn.softplus(raw)).astype(jnp.int32)

    s_past = _shift(past_shift_raw)
    s_future = _shift(future_shift_raw)
    t = jnp.arange(_T)
    src_f = jnp.clip(t - s_future, 0, _T - 1)
    src_p = jnp.clip(t + s_past, 0, _T - 1)
    onehot = jnp.stack([
        (src_f[:, None] == t[None, :]).astype(jnp.float32),
        (src_p[:, None] == t[None, :]).astype(jnp.float32),
    ])

    head = pl.pallas_call(
        _body,
        grid=(B_T // _T,),
        in_specs=[
            pl.BlockSpec((2, _T, _T), lambda b: (0, 0, 0)),
            pl.BlockSpec((_T, 8, C), lambda b: (b, 0, 0)),
        ],
        out_specs=pl.BlockSpec((_T, 8, C), lambda b: (b, 0, 0)),
        out_shape=jax.ShapeDtypeStruct((B_T, 8, C), x.dtype),
    )(onehot, x)

    n_idx = jax.lax.broadcasted_iota(jnp.int32, (1, N, 1), 1)
    return jnp.where(n_idx == 0, head[:, 0:1, :], x)


# speedup vs baseline: 1.9860x; 1.9860x over previous
import jax
import jax.numpy as jnp
from jax.experimental import pallas as pl
from jax.experimental.pallas import tpu as pltpu

_T = 16


def _body(mf_ref, mp_ref, cls_ref, o_ref):
    C = cls_ref.shape[-1]
    cls = cls_ref[...]
    shifted_f = jnp.dot(mf_ref[...], cls, preferred_element_type=jnp.float32)
    shifted_p = jnp.dot(mp_ref[...], cls, preferred_element_type=jnp.float32)
    c_idx = jax.lax.broadcasted_iota(jnp.int32, cls.shape, 1)
    o_ref[...] = cls + jnp.where(c_idx < C // 2, shifted_f, shifted_p)


def kernel(x, past_shift_raw, future_shift_raw):
    B_T, N, C = x.shape

    def _shift(raw):
        return jnp.round(jax.nn.softplus(raw)).astype(jnp.int32)

    s_past = _shift(past_shift_raw)
    s_future = _shift(future_shift_raw)
    t = jnp.arange(_T)
    src_f = jnp.clip(t - s_future, 0, _T - 1)
    src_p = jnp.clip(t + s_past, 0, _T - 1)
    eye_b = jnp.eye(B_T // _T, dtype=jnp.float32)
    mf = jnp.kron(eye_b, (src_f[:, None] == t[None, :]).astype(jnp.float32))
    mp = jnp.kron(eye_b, (src_p[:, None] == t[None, :]).astype(jnp.float32))

    new_cls = pl.pallas_call(
        _body,
        in_specs=[
            pl.BlockSpec(memory_space=pltpu.VMEM),
            pl.BlockSpec(memory_space=pltpu.VMEM),
            pl.BlockSpec(memory_space=pltpu.VMEM),
        ],
        out_specs=pl.BlockSpec(memory_space=pltpu.VMEM),
        out_shape=jax.ShapeDtypeStruct((B_T, C), x.dtype),
    )(mf, mp, x[:, 0, :])

    n_idx = jax.lax.broadcasted_iota(jnp.int32, (1, N, 1), 1)
    return jnp.where(n_idx == 0, new_cls[:, None, :], x)


# speedup vs baseline: 1.9872x; 1.0006x over previous
import jax
import jax.numpy as jnp
from jax.experimental import pallas as pl
from jax.experimental.pallas import tpu as pltpu

_T = 16


def _body(mf_ref, mp_ref, cls_ref, o_ref):
    C = cls_ref.shape[-1]
    g2 = cls_ref[:, :C // 2]
    g1 = cls_ref[:, C // 2:]
    o_ref[:, :C // 2] = g2 + jnp.dot(
        mf_ref[...], g2, preferred_element_type=jnp.float32)
    o_ref[:, C // 2:] = g1 + jnp.dot(
        mp_ref[...], g1, preferred_element_type=jnp.float32)


def kernel(x, past_shift_raw, future_shift_raw):
    B_T, N, C = x.shape

    def _shift(raw):
        return jnp.round(jax.nn.softplus(raw)).astype(jnp.int32)

    s_past = _shift(past_shift_raw)
    s_future = _shift(future_shift_raw)
    t = jnp.arange(_T)
    src_f = jnp.clip(t - s_future, 0, _T - 1)
    src_p = jnp.clip(t + s_past, 0, _T - 1)
    eye_b = jnp.eye(B_T // _T, dtype=jnp.float32)
    mf = jnp.kron(eye_b, (src_f[:, None] == t[None, :]).astype(jnp.float32))
    mp = jnp.kron(eye_b, (src_p[:, None] == t[None, :]).astype(jnp.float32))

    new_cls = pl.pallas_call(
        _body,
        in_specs=[
            pl.BlockSpec(memory_space=pltpu.VMEM),
            pl.BlockSpec(memory_space=pltpu.VMEM),
            pl.BlockSpec(memory_space=pltpu.VMEM),
        ],
        out_specs=pl.BlockSpec(memory_space=pltpu.VMEM),
        out_shape=jax.ShapeDtypeStruct((B_T, C), x.dtype),
    )(mf, mp, x[:, 0, :])

    n_idx = jax.lax.broadcasted_iota(jnp.int32, (1, N, 1), 1)
    return jnp.where(n_idx == 0, new_cls[:, None, :], x)


# speedup vs baseline: 2.0083x; 1.0106x over previous
import functools
import jax
import jax.numpy as jnp
from jax import lax
from jax.experimental import pallas as pl
from jax.experimental.pallas import tpu as pltpu, tpu_sc as plsc

_T = 16


def _sc_cls(B_T, C):
    mesh = plsc.VectorSubcoreMesh(core_axis_name="c", subcore_axis_name="s")

    @functools.partial(
        pl.kernel,
        mesh=mesh,
        out_type=jax.ShapeDtypeStruct((B_T, C), jnp.float32),
        scratch_types=[
            pltpu.VMEM((_T, C), jnp.float32),
            pltpu.VMEM((_T, C), jnp.float32),
            pltpu.VMEM((_T, C), jnp.float32),
            pltpu.VMEM((_T, C), jnp.float32),
            pltpu.VMEM((2, _T), jnp.int32),
            pltpu.SemaphoreType.DMA,
            pltpu.SemaphoreType.DMA,
            pltpu.SemaphoreType.DMA,
        ],
    )
    def k(cls_hbm, srcs_hbm, out_hbm, buf_in, buf_out, gath_f, gath_p,
          srcs_vm, sem, sem_f, sem_p):
        wid = lax.axis_index("s") * 2 + lax.axis_index("c")
        base = pl.multiple_of(wid * _T, _T)

        cp = pltpu.make_async_copy(srcs_hbm, srcs_vm, sem)
        cp.start()
        cp.wait()
        cp_in = pltpu.make_async_copy(
            cls_hbm.at[pl.ds(base, _T), :], buf_in, sem)
        cp_in.start()

        idx_f = srcs_vm[0, :] + base
        idx_p = srcs_vm[1, :] + base
        cp_f = pltpu.make_async_copy(cls_hbm.at[idx_f], gath_f, sem_f)
        cp_f.start()
        cp_p = pltpu.make_async_copy(cls_hbm.at[idx_p], gath_p, sem_p)
        cp_p.start()
        cp_in.wait()
        cp_f.wait()
        cp_p.wait()

        for t in range(_T):
            for j in range(C // 16):
                g = gath_f if j < (C // 2) // 16 else gath_p
                buf_out[t, pl.ds(j * 16, 16)] = (
                    buf_in[t, pl.ds(j * 16, 16)]
                    + g[t, pl.ds(j * 16, 16)])

        cp = pltpu.make_async_copy(buf_out, out_hbm.at[pl.ds(base, _T), :], sem)
        cp.start()
        cp.wait()

    return k


def kernel(x, past_shift_raw, future_shift_raw):
    B_T, N, C = x.shape

    def _shift(raw):
        return jnp.round(jax.nn.softplus(raw)).astype(jnp.int32)

    s_past = _shift(past_shift_raw)
    s_future = _shift(future_shift_raw)
    t = jnp.arange(_T)
    src_f = jnp.clip(t - s_future, 0, _T - 1)
    src_p = jnp.clip(t + s_past, 0, _T - 1)
    srcs = jnp.stack([src_f, src_p]).astype(jnp.int32)

    new_cls = _sc_cls(B_T, C)(x[:, 0, :], srcs)

    n_idx = jax.lax.broadcasted_iota(jnp.int32, (1, N, 1), 1)
    return jnp.where(n_idx == 0, new_cls[:, None, :], x)
